# baseline (speedup 1.0000x reference)
import functools

import jax
import jax.numpy as jnp
from jax import lax
from jax.experimental import pallas as pl
from jax.experimental.pallas import tpu as pltpu
from jax.experimental.pallas import tpu_sc as plsc

NUM_VARIATES = 100000
D_MODEL = 128
B, T = 4096, 200
B_FLAT = B * T

_info = plsc.get_sparse_core_info()
NC, NS = _info.num_cores, _info.num_subcores
NW = NC * NS
ROWS_PER_W = B_FLAT // NW
CHUNK = 128
N_CHUNKS = ROWS_PER_W // CHUNK
BUFS = 5
LAG = 2


def _make_gather():
    mesh = plsc.VectorSubcoreMesh(core_axis_name="c", subcore_axis_name="s")

    @functools.partial(
        pl.kernel,
        mesh=mesh,
        out_type=jax.ShapeDtypeStruct((B_FLAT, D_MODEL), jnp.float32),
        scratch_types=[
            pltpu.VMEM((N_CHUNKS, CHUNK), jnp.int32),
            pltpu.VMEM((BUFS, CHUNK, D_MODEL), jnp.float32),
            pltpu.SemaphoreType.DMA((BUFS,)),
            pltpu.SemaphoreType.DMA((BUFS,)),
        ],
    )
    def gather_kernel(idx_hbm, table_hbm, out_hbm, idx_v, rows, gsem, ssem):
        wid = lax.axis_index("s") * NC + lax.axis_index("c")
        base = wid * ROWS_PER_W

        pltpu.sync_copy(idx_hbm.at[pl.ds(wid * N_CHUNKS, N_CHUNKS)], idx_v)

        def fire_gather(b, k):
            pltpu.async_copy(table_hbm.at[idx_v.at[k]], rows.at[b],
                             gsem.at[b])

        def wait_gather(b):
            pltpu.make_async_copy(table_hbm.at[pl.ds(0, CHUNK)], rows.at[b],
                                  gsem.at[b]).wait()

        def fire_store(b, k):
            pltpu.async_copy(rows.at[b],
                             out_hbm.at[pl.ds(base + k * CHUNK, CHUNK)],
                             ssem.at[b])

        def wait_store(b):
            pltpu.make_async_copy(rows.at[b], out_hbm.at[pl.ds(base, CHUNK)],
                                  ssem.at[b]).wait()

        for b in range(BUFS):
            fire_gather(b, b)

        @pl.loop(0, N_CHUNKS, step=BUFS)
        def _round(k0):
            for b in range(BUFS):
                k = k0 + b
                wait_gather(b)
                fire_store(b, k)

                bj = (b - LAG) % BUFS
                j = k - LAG

                @pl.when(jnp.logical_and(j >= 0, j + BUFS < N_CHUNKS))
                def _():
                    wait_store(bj)
                    fire_gather(bj, j + BUFS)

        for b in range(BUFS):
            wait_store(b)

    return gather_kernel


_gather = _make_gather()


@jax.jit
def kernel(variate_ids, embed_table):
    idx = variate_ids.reshape(B_FLAT // CHUNK, CHUNK).astype(jnp.int32)
    out = _gather(idx, embed_table)
    return out.reshape(B, T, D_MODEL)
